# Initial kernel scaffold; baseline (speedup 1.0000x reference)
import jax
import jax.numpy as jnp
from jax.experimental import pallas as pl


def kernel(inputs_in, labels_in, kernel, bias):
    raise NotImplementedError("write your pallas kernel here")


# speedup vs baseline: 1.5393x; 1.5393x over previous
import functools

import jax
import jax.numpy as jnp
from jax import lax
from jax.experimental import pallas as pl
from jax.experimental.pallas import tpu as pltpu
from jax.experimental.pallas import tpu_sc as plsc

NUM_SAMPLED = 1024
NUM_CLASSES = 100000
LAMB = 0.1
BATCH = 4096
DIM = 128

_NW = 32
_TB = BATCH // _NW
_SB = NUM_SAMPLED // _NW


def _sc_gather(table, bias, labels, sampled):
    mesh = plsc.VectorSubcoreMesh(core_axis_name="c", subcore_axis_name="s")

    @functools.partial(
        pl.kernel,
        out_type=(
            jax.ShapeDtypeStruct((BATCH, DIM), jnp.float32),
            jax.ShapeDtypeStruct((BATCH,), jnp.float32),
            jax.ShapeDtypeStruct((NUM_SAMPLED, DIM), jnp.float32),
            jax.ShapeDtypeStruct((NUM_SAMPLED,), jnp.float32),
        ),
        mesh=mesh,
        scratch_types=[
            pltpu.VMEM((_TB,), jnp.int32),
            pltpu.VMEM((_SB,), jnp.int32),
            pltpu.VMEM((_TB, DIM), jnp.float32),
            pltpu.VMEM((_SB, DIM), jnp.float32),
            pltpu.VMEM((_TB,), jnp.float32),
            pltpu.VMEM((_SB,), jnp.float32),
            pltpu.SemaphoreType.DMA,
            pltpu.SemaphoreType.DMA,
            pltpu.SemaphoreType.DMA,
            pltpu.SemaphoreType.DMA,
        ],
    )
    def k(table_h, bias_h, labels_h, sampled_h,
          tw_h, tb_h, sw_h, sb_h,
          lidx, sidx, twv, swv, tbv, sbv, sem1, sem2, sem3, sem4):
        wid = lax.axis_index("s") * 2 + lax.axis_index("c")
        tbase = wid * _TB
        sbase = wid * _SB
        pltpu.sync_copy(labels_h.at[pl.ds(tbase, _TB)], lidx)
        pltpu.sync_copy(sampled_h.at[pl.ds(sbase, _SB)], sidx)
        c1 = pltpu.async_copy(table_h.at[lidx], twv, sem1)
        c2 = pltpu.async_copy(table_h.at[sidx], swv, sem2)
        c3 = pltpu.async_copy(bias_h.at[lidx], tbv, sem3)
        c4 = pltpu.async_copy(bias_h.at[sidx], sbv, sem4)
        c1.wait()
        pltpu.sync_copy(twv, tw_h.at[pl.ds(tbase, _TB)])
        c2.wait()
        pltpu.sync_copy(swv, sw_h.at[pl.ds(sbase, _SB)])
        c3.wait()
        pltpu.sync_copy(tbv, tb_h.at[pl.ds(tbase, _TB)])
        c4.wait()
        pltpu.sync_copy(sbv, sb_h.at[pl.ds(sbase, _SB)])

    return k(table, bias, labels, sampled)


_BB = 512
_LOGNC1 = float(jnp.log(jnp.float32(NUM_CLASSES + 1.0)))


def _tc_body(x_ref, tw_ref, tb_ref, lab_ref, sw_ref, sb_ref, samp_ref,
             out_ref):
    i = pl.program_id(0)
    x = x_ref[...]
    s_log = lax.dot_general(x, sw_ref[...], (((1,), (1,)), ((), ())),
                            preferred_element_type=jnp.float32)
    samp_f = samp_ref[...]
    sp = (jnp.log(samp_f + 2.0) - jnp.log(samp_f + 1.0)) / _LOGNC1
    sq = 1.0 - jnp.exp(NUM_SAMPLED * jnp.log(1.0 - sp))
    s_log = s_log + (sb_ref[...] - jnp.log(sq + 1e-20))
    hit = lab_ref[...] == samp_ref[...]
    s_log = jnp.where(hit, s_log - 1e9, s_log)
    labf = lab_ref[...]
    tp = (jnp.log(labf + 2.0) - jnp.log(labf + 1.0)) / _LOGNC1
    tq = 1.0 - jnp.exp(NUM_SAMPLED * jnp.log(1.0 - tp))
    t_log = (jnp.sum(x * tw_ref[...], axis=1, keepdims=True)
             + tb_ref[...] - jnp.log(tq + 1e-20))
    m = jnp.maximum(jnp.max(s_log, axis=1, keepdims=True), t_log)
    se = jnp.sum(jnp.exp(s_log - m), axis=1, keepdims=True) + jnp.exp(t_log - m)
    loss = jnp.log(se) + m - t_log

    @pl.when(i == 0)
    def _():
        out_ref[0, 0] = 0.0

    out_ref[0, 0] += jnp.sum(loss)

    @pl.when(i == BATCH // _BB - 1)
    def _():
        out_ref[0, 0] *= jnp.float32(LAMB / BATCH)


def _tc_loss(x, tw, tb_col, lab_colf, sw, sb_row, samp_rowf):
    grid = BATCH // _BB
    return pl.pallas_call(
        _tc_body,
        grid=(grid,),
        in_specs=[
            pl.BlockSpec((_BB, DIM), lambda i: (i, 0)),
            pl.BlockSpec((_BB, DIM), lambda i: (i, 0)),
            pl.BlockSpec((_BB, 1), lambda i: (i, 0)),
            pl.BlockSpec((_BB, 1), lambda i: (i, 0)),
            pl.BlockSpec((NUM_SAMPLED, DIM), lambda i: (0, 0)),
            pl.BlockSpec((1, NUM_SAMPLED), lambda i: (0, 0)),
            pl.BlockSpec((1, NUM_SAMPLED), lambda i: (0, 0)),
        ],
        out_specs=pl.BlockSpec(memory_space=pltpu.SMEM),
        out_shape=jax.ShapeDtypeStruct((1, 1), jnp.float32),
    )(x, tw, tb_col, lab_colf, sw, sb_row, samp_rowf)


def _sampled_ids():
    key = jax.random.fold_in(jax.random.key(42), 5)
    u = jax.random.uniform(key, (NUM_SAMPLED,))
    s = jnp.floor(jnp.exp(u * jnp.log(NUM_CLASSES + 1.0))) - 1.0
    return jnp.clip(s, 0, NUM_CLASSES - 1).astype(jnp.int32)


def kernel(inputs_in, labels_in, kernel, bias):
    x = inputs_in[:, 2, :].astype(jnp.float32)
    labels = labels_in[:, 3].astype(jnp.int32)
    sampled = _sampled_ids()
    tw, tb, sw, sb = _sc_gather(kernel, bias, labels, sampled)
    out = _tc_loss(
        x, tw,
        tb.reshape(BATCH, 1),
        labels.astype(jnp.float32).reshape(BATCH, 1),
        sw,
        sb.reshape(1, NUM_SAMPLED),
        sampled.astype(jnp.float32).reshape(1, NUM_SAMPLED),
    )
    return out[0, 0]
